# Initial kernel scaffold; baseline (speedup 1.0000x reference)
import jax
import jax.numpy as jnp
from jax.experimental import pallas as pl


def kernel(x, edge_index, lin1_W, lin1_b, conv1_W, conv1_phiW, conv1_b, lin2_W, lin2_b, conv2_W, conv2_phiW, conv2_b):
    raise NotImplementedError("write your pallas kernel here")


# speedup vs baseline: 15.8240x; 15.8240x over previous
import functools
import math

import jax
import jax.numpy as jnp
from jax import lax
from jax.experimental import pallas as pl
from jax.experimental.pallas import tpu as pltpu
from jax.experimental.pallas import tpu_sc as plsc

_NC, _NS = 2, 16
_N_CLS = 40
_CP = 48
_R = 1000


def _mesh():
    return plsc.VectorSubcoreMesh(core_axis_name="c", subcore_axis_name="s")


def _make_deg(n_rows_pad, n_chunks, rows_per_tile):
    @functools.partial(
        pl.kernel,
        out_type=jax.ShapeDtypeStruct((_NC, n_rows_pad, 1), jnp.float32),
        mesh=_mesh(),
        scratch_types=[
            pltpu.VMEM((n_chunks, 128), jnp.int32),
            pltpu.VMEM((128, 1), jnp.float32),
            pltpu.VMEM((128, 1), jnp.float32),
            pltpu.VMEM_SHARED((n_rows_pad, 1), jnp.float32),
        ],
        compiler_params=pltpu.CompilerParams(use_tc_tiling_on_sc=False),
    )
    def k(dst_hbm, ones_hbm, zeros_hbm, out_hbm, dst_v, ones_v, zbuf, acc):
        c = lax.axis_index("c")
        s = lax.axis_index("s")
        wid = c * _NS + s
        pltpu.sync_copy(dst_hbm.at[wid], dst_v)
        pltpu.sync_copy(ones_hbm, ones_v)
        pltpu.sync_copy(zeros_hbm, zbuf)
        base = s * rows_per_tile
        for kk in range(rows_per_tile // 128):
            pltpu.sync_copy(zbuf, acc.at[pl.ds(base + kk * 128, 128)])
        plsc.subcore_barrier()

        def body(j, carry):
            pltpu.sync_copy(ones_v, acc.at[dst_v.at[j]], add=True)
            return carry

        lax.fori_loop(0, n_chunks, body, 0)
        plsc.subcore_barrier()
        for kk in range(rows_per_tile // 128):
            pltpu.sync_copy(acc.at[pl.ds(base + kk * 128, 128)], zbuf)
            pltpu.sync_copy(zbuf, out_hbm.at[c, pl.ds(base + kk * 128, 128)])

    return k


def _make_edge_scatter(n_rows_pad, width, n_chunks, rows_per_tile):
    @functools.partial(
        pl.kernel,
        out_type=jax.ShapeDtypeStruct((_NC, n_rows_pad, width), jnp.float32),
        mesh=_mesh(),
        scratch_types=[
            pltpu.VMEM((n_chunks, 128), jnp.int32),
            pltpu.VMEM((n_chunks, 128), jnp.int32),
            pltpu.VMEM((128, width), jnp.float32),
            pltpu.VMEM_SHARED((n_rows_pad, width), jnp.float32),
            pltpu.SemaphoreType.DMA,
        ],
        compiler_params=pltpu.CompilerParams(use_tc_tiling_on_sc=False),
    )
    def k(y_hbm, src_hbm, dst_hbm, zeros_hbm, out_hbm, src_v, dst_v, buf, acc, sem):
        c = lax.axis_index("c")
        s = lax.axis_index("s")
        wid = c * _NS + s
        pltpu.sync_copy(src_hbm.at[wid], src_v)
        pltpu.sync_copy(dst_hbm.at[wid], dst_v)
        pltpu.sync_copy(zeros_hbm, buf)
        base = s * rows_per_tile
        for kk in range(rows_per_tile // 128):
            pltpu.sync_copy(buf, acc.at[pl.ds(base + kk * 128, 128)])
        plsc.subcore_barrier()

        def body(j, carry):
            pltpu.async_copy(y_hbm.at[src_v.at[j]], buf, sem).wait()
            pltpu.sync_copy(buf, acc.at[dst_v.at[j]], add=True)
            return carry

        lax.fori_loop(0, n_chunks, body, 0)
        plsc.subcore_barrier()
        for kk in range(rows_per_tile // 128):
            pltpu.sync_copy(acc.at[pl.ds(base + kk * 128, 128)], buf)
            pltpu.sync_copy(buf, out_hbm.at[c, pl.ds(base + kk * 128, 128)])

    return k


def _tc1_body(x_ref, w1t_ref, b1_ref, phi1t_ref, deg_ref, h_ref, y_ref):
    h = jnp.maximum(
        jnp.dot(x_ref[...], w1t_ref[...], preferred_element_type=jnp.float32)
        + b1_ref[...], 0.0)
    d = deg_ref[...]
    dis = lax.rsqrt(d[0] + d[1] + 1.0)
    y_ref[...] = jnp.dot(h, phi1t_ref[...], preferred_element_type=jnp.float32) * dis
    h_ref[...] = h


def _tc1(x, w1t, b1, phi1t, degp):
    n, dd = x.shape
    npad = degp.shape[1]
    return pl.pallas_call(
        _tc1_body,
        grid=(n // _R,),
        in_specs=[
            pl.BlockSpec((_R, dd), lambda i: (i, 0)),
            pl.BlockSpec((dd, dd), lambda i: (0, 0)),
            pl.BlockSpec((1, dd), lambda i: (0, 0)),
            pl.BlockSpec((dd, dd), lambda i: (0, 0)),
            pl.BlockSpec((_NC, _R, 1), lambda i: (0, i, 0)),
        ],
        out_specs=[pl.BlockSpec((_R, dd), lambda i: (i, 0))] * 2,
        out_shape=[jax.ShapeDtypeStruct((n, dd), jnp.float32)] * 2,
    )(x, w1t, b1, phi1t, degp)


def _tc2_body(h_ref, y1_ref, z_ref, deg_ref, aw1t_ref, cb1_ref, w2t_ref,
              b2_ref, phi2t_ref, g_ref, y2_ref):
    d = deg_ref[...]
    dis = lax.rsqrt(d[0] + d[1] + 1.0)
    z = z_ref[...]
    h = h_ref[...]
    gcn = (z[0] + z[1] + y1_ref[...]) * dis
    h2 = h + 0.1 * jnp.tanh(
        jnp.dot(h, aw1t_ref[...], preferred_element_type=jnp.float32)
        + gcn + cb1_ref[...])
    g = jnp.dot(h2, w2t_ref[...], preferred_element_type=jnp.float32) + b2_ref[...]
    y2_ref[...] = jnp.dot(g, phi2t_ref[...], preferred_element_type=jnp.float32) * dis
    g_ref[...] = g


def _tc2(h, y1, z1, degp, aw1t, cb1, w2t, b2, phi2t):
    n, dd = h.shape
    return pl.pallas_call(
        _tc2_body,
        grid=(n // _R,),
        in_specs=[
            pl.BlockSpec((_R, dd), lambda i: (i, 0)),
            pl.BlockSpec((_R, dd), lambda i: (i, 0)),
            pl.BlockSpec((_NC, _R, dd), lambda i: (0, i, 0)),
            pl.BlockSpec((_NC, _R, 1), lambda i: (0, i, 0)),
            pl.BlockSpec((dd, dd), lambda i: (0, 0)),
            pl.BlockSpec((1, dd), lambda i: (0, 0)),
            pl.BlockSpec((dd, _CP), lambda i: (0, 0)),
            pl.BlockSpec((1, _CP), lambda i: (0, 0)),
            pl.BlockSpec((_CP, _CP), lambda i: (0, 0)),
        ],
        out_specs=[pl.BlockSpec((_R, _CP), lambda i: (i, 0))] * 2,
        out_shape=[jax.ShapeDtypeStruct((n, _CP), jnp.float32)] * 2,
    )(h, y1, z1, degp, aw1t, cb1, w2t, b2, phi2t)


def _tc3_body(g_ref, y2_ref, z_ref, deg_ref, aw2t_ref, cb2_ref, o_ref):
    d = deg_ref[...]
    dis = lax.rsqrt(d[0] + d[1] + 1.0)
    z = z_ref[...]
    g = g_ref[...]
    gcn = (z[0] + z[1] + y2_ref[...]) * dis
    g2 = g + 0.1 * jnp.tanh(
        jnp.dot(g, aw2t_ref[...], preferred_element_type=jnp.float32)
        + gcn + cb2_ref[...])
    col = lax.broadcasted_iota(jnp.int32, g2.shape, 1)
    valid = col < _N_CLS
    m = jnp.max(jnp.where(valid, g2, -jnp.inf), axis=1, keepdims=True)
    e = jnp.where(valid, jnp.exp(g2 - m), 0.0)
    lse = m + jnp.log(jnp.sum(e, axis=1, keepdims=True))
    o_ref[...] = (g2 - lse)[:, :_N_CLS]


def _tc3(g, y2, z2, degp, aw2t, cb2):
    n = g.shape[0]
    return pl.pallas_call(
        _tc3_body,
        grid=(n // _R,),
        in_specs=[
            pl.BlockSpec((_R, _CP), lambda i: (i, 0)),
            pl.BlockSpec((_R, _CP), lambda i: (i, 0)),
            pl.BlockSpec((_NC, _R, _CP), lambda i: (0, i, 0)),
            pl.BlockSpec((_NC, _R, 1), lambda i: (0, i, 0)),
            pl.BlockSpec((_CP, _CP), lambda i: (0, 0)),
            pl.BlockSpec((1, _CP), lambda i: (0, 0)),
        ],
        out_specs=pl.BlockSpec((_R, _N_CLS), lambda i: (i, 0)),
        out_shape=jax.ShapeDtypeStruct((n, _N_CLS), jnp.float32),
    )(g, y2, z2, degp, aw2t, cb2)


def kernel(x, edge_index, lin1_W, lin1_b, conv1_W, conv1_phiW, conv1_b,
           lin2_W, lin2_b, conv2_W, conv2_phiW, conv2_b):
    n, dd = x.shape
    e = edge_index.shape[1]
    nw = _NC * _NS
    cpt = math.ceil(e / (nw * 128))
    ep = nw * 128 * cpt
    rpt = 128 * math.ceil((n + 1) / (_NS * 128))
    npad = _NS * rpt

    src = edge_index[0].astype(jnp.int32)
    dst = edge_index[1].astype(jnp.int32)
    pad = ep - e
    srcp = jnp.concatenate([src, jnp.zeros((pad,), jnp.int32)]).reshape(nw, cpt, 128)
    dstp = jnp.concatenate([dst, jnp.full((pad,), n, jnp.int32)]).reshape(nw, cpt, 128)

    w1t = lin1_W.T
    b1 = lin1_b.reshape(1, dd)
    phi1t = conv1_phiW.T
    aw1t = conv1_W.T - conv1_W - 0.1 * jnp.eye(dd, dtype=jnp.float32)
    cb1 = conv1_b.reshape(1, dd)
    w2t = jnp.zeros((dd, _CP), jnp.float32).at[:, :_N_CLS].set(lin2_W.T)
    b2 = jnp.zeros((1, _CP), jnp.float32).at[0, :_N_CLS].set(lin2_b)
    phi2t = jnp.zeros((_CP, _CP), jnp.float32).at[:_N_CLS, :_N_CLS].set(conv2_phiW.T)
    aw2t = jnp.zeros((_CP, _CP), jnp.float32).at[:_N_CLS, :_N_CLS].set(
        conv2_W.T - conv2_W - 0.1 * jnp.eye(_N_CLS, dtype=jnp.float32))
    cb2 = jnp.zeros((1, _CP), jnp.float32).at[0, :_N_CLS].set(conv2_b)

    ones1 = jnp.ones((128, 1), jnp.float32)
    zeros1 = jnp.zeros((128, 1), jnp.float32)
    zerosD = jnp.zeros((128, dd), jnp.float32)
    zerosC = jnp.zeros((128, _CP), jnp.float32)

    degp = _make_deg(npad, cpt, rpt)(dstp, ones1, zeros1)
    h, y1 = _tc1(x, w1t, b1, phi1t, degp)
    z1 = _make_edge_scatter(npad, dd, cpt, rpt)(y1, srcp, dstp, zerosD)
    g, y2 = _tc2(h, y1, z1, degp, aw1t, cb1, w2t, b2, phi2t)
    z2 = _make_edge_scatter(npad, _CP, cpt, rpt)(y2, srcp, dstp, zerosC)
    return _tc3(g, y2, z2, degp, aw2t, cb2)
